# baseline (speedup 1.0000x reference)
import functools

import jax
import jax.numpy as jnp
from jax import lax
from jax.experimental import pallas as pl
from jax.experimental.pallas import tpu as pltpu
from jax.experimental.pallas import tpu_sc as plsc

NC = 2
NS = 16
NW = NC * NS
CH = 128
K = 8


def _sc_mesh():
  return plsc.VectorSubcoreMesh(core_axis_name="c", subcore_axis_name="s")


def _make_deg_kernel(npad, epw):
  rpt = npad // NS
  w = 16
  nslabs = epw // (K * CH)
  assert nslabs % 2 == 0 and nslabs >= 2

  @functools.partial(
      pl.kernel,
      out_type=jax.ShapeDtypeStruct((NC * npad, w), jnp.float32),
      mesh=_sc_mesh(),
      scratch_types=[
          pltpu.VMEM((K, CH), jnp.int32),
          pltpu.VMEM((K, CH), jnp.int32),
          pltpu.VMEM((CH, w), jnp.float32),
          pltpu.VMEM_SHARED((npad, w), jnp.float32),
          pltpu.SemaphoreType.DMA,
          pltpu.SemaphoreType.DMA,
      ],
      compiler_params=pltpu.CompilerParams(use_tc_tiling_on_sc=False),
  )
  def deg_kernel(dst2_hbm, ones_hbm, zrow_hbm, out_hbm,
                 dst_s0, dst_s1, ones_v, acc, ssem0, ssem1):
    cid = lax.axis_index("c")
    sid = lax.axis_index("s")
    wid = sid * NC + cid
    r0 = wid * (epw // CH)

    pltpu.sync_copy(zrow_hbm, acc.at[pl.ds(sid * rpt, rpt)])
    pltpu.sync_copy(ones_hbm, ones_v)
    plsc.subcore_barrier()

    def load(buf, s):
      pltpu.sync_copy(dst2_hbm.at[pl.ds(r0 + s * K, K)], buf)

    def fire(buf, sem):
      for k in range(K):
        pltpu.async_copy(ones_v, acc.at[buf.at[k]], sem, add=True)

    def drain(sem):
      for _ in range(K):
        pltpu.make_async_copy(ones_hbm, ones_v, sem).wait()

    load(dst_s0, 0)

    def body(j, carry):
      s = 2 * j
      fire(dst_s0, ssem0)

      @pl.when(j > 0)
      def _():
        drain(ssem1)

      load(dst_s1, s + 1)
      fire(dst_s1, ssem1)
      drain(ssem0)

      @pl.when(j + 1 < nslabs // 2)
      def _():
        load(dst_s0, s + 2)

      return carry

    lax.fori_loop(0, nslabs // 2, body, 0)
    drain(ssem1)
    plsc.subcore_barrier()
    pltpu.sync_copy(acc.at[pl.ds(sid * rpt, rpt)],
                    out_hbm.at[pl.ds(cid * npad + sid * rpt, rpt)])

  return deg_kernel


def _make_agg_kernel(npad, epw0, epw1, w, kc, nbuf):
  rpt = npad // NS
  ns0 = epw0 // (kc * CH)
  ns1 = epw1 // (kc * CH)
  for nslabs in (ns0, ns1):
    assert nslabs % nbuf == 0 and nslabs >= 2 * nbuf

  @functools.partial(
      pl.kernel,
      out_type=jax.ShapeDtypeStruct((NC * npad, w), jnp.float32),
      mesh=_sc_mesh(),
      scratch_types=(
          [pltpu.VMEM((kc, CH), jnp.int32) for _ in range(nbuf)] +
          [pltpu.VMEM((kc, CH), jnp.int32) for _ in range(nbuf)] +
          [pltpu.VMEM((kc * CH, w), jnp.float32) for _ in range(nbuf)] +
          [pltpu.VMEM_SHARED((npad, w), jnp.float32)] +
          [pltpu.SemaphoreType.DMA for _ in range(2 * nbuf)]
      ),
      compiler_params=pltpu.CompilerParams(use_tc_tiling_on_sc=False),
  )
  def agg_kernel(src2_hbm, dst2_hbm, g_hbm, zrow_hbm, out_hbm, *bufs):
    src_s = bufs[0:nbuf]
    dst_s = bufs[nbuf:2 * nbuf]
    rows = bufs[2 * nbuf:3 * nbuf]
    acc = bufs[3 * nbuf]
    gsem = bufs[3 * nbuf + 1:3 * nbuf + 1 + nbuf]
    ssem = bufs[3 * nbuf + 1 + nbuf:3 * nbuf + 1 + 2 * nbuf]

    cid = lax.axis_index("c")
    sid = lax.axis_index("s")
    cw = jnp.where(cid == 0, epw0 // CH, epw1 // CH)
    nslabs = jnp.where(cid == 0, ns0, ns1)
    r0 = cid * (NS * (epw0 // CH)) + sid * cw

    pltpu.sync_copy(zrow_hbm, acc.at[pl.ds(sid * rpt, rpt)])
    plsc.subcore_barrier()

    def fire_gathers(b, s):
      pltpu.sync_copy(src2_hbm.at[pl.ds(r0 + s * kc, kc)], src_s[b])
      pltpu.sync_copy(dst2_hbm.at[pl.ds(r0 + s * kc, kc)], dst_s[b])
      for k in range(kc):
        pltpu.async_copy(g_hbm.at[src_s[b].at[k]],
                         rows[b].at[pl.ds(k * CH, CH)], gsem[b])

    def drain_gathers(b):
      pltpu.make_async_copy(g_hbm.at[pl.ds(0, kc * CH)], rows[b],
                            gsem[b]).wait()

    def fire_scatters(b):
      for k in range(kc):
        pltpu.async_copy(rows[b].at[pl.ds(k * CH, CH)],
                         acc.at[dst_s[b].at[k]], ssem[b], add=True)

    def drain_scatters(b):
      pltpu.make_async_copy(g_hbm.at[pl.ds(0, kc * CH)], rows[b],
                            ssem[b]).wait()

    for b in range(nbuf - 1):
      fire_gathers(b, b)

    def body(j, carry):
      for r in range(nbuf):
        s = nbuf * j + r
        b2 = (r + nbuf - 1) % nbuf

        @pl.when(s + nbuf - 1 < nslabs)
        def _():
          if r == 0:
            @pl.when(j > 0)
            def _():
              drain_scatters(b2)
          else:
            drain_scatters(b2)
          fire_gathers(b2, s + nbuf - 1)

        drain_gathers(r)
        fire_scatters(r)
      return carry

    lax.fori_loop(0, nslabs // nbuf, body, 0)
    for b in range(nbuf):
      drain_scatters(b)
    plsc.subcore_barrier()
    pltpu.sync_copy(acc.at[pl.ds(sid * rpt, rpt)],
                    out_hbm.at[pl.ds(cid * npad + sid * rpt, rpt)])

  return agg_kernel


def _matmul_xw0(x, w0, bm):
  n, d = x.shape
  h = w0.shape[1]

  def body(x_ref, w_ref, o_ref):
    o_ref[...] = jnp.dot(x_ref[...], w_ref[...],
                         preferred_element_type=jnp.float32)

  return pl.pallas_call(
      body,
      grid=(n // bm,),
      in_specs=[
          pl.BlockSpec((bm, d), lambda i: (i, 0)),
          pl.BlockSpec((d, h), lambda i: (0, 0)),
      ],
      out_specs=pl.BlockSpec((bm, h), lambda i: (i, 0)),
      out_shape=jax.ShapeDtypeStruct((n, h), jnp.float32),
  )(x, w0)


def _prep_g0(degp, xw0, npad, bm):
  n, h = xw0.shape

  def body(d_ref, x_ref, dinv_ref, g_ref):
    deg = d_ref[0, :, 0:1] + d_ref[1, :, 0:1] + 1.0
    dinv = lax.rsqrt(deg)
    dinv_b = jnp.broadcast_to(dinv, (bm, h))
    dinv_ref[...] = dinv_b
    g_ref[...] = dinv_b * x_ref[...]

  return pl.pallas_call(
      body,
      grid=(n // bm,),
      in_specs=[
          pl.BlockSpec((2, bm, 16), lambda i: (0, i, 0)),
          pl.BlockSpec((bm, h), lambda i: (i, 0)),
      ],
      out_specs=[
          pl.BlockSpec((bm, h), lambda i: (i, 0)),
          pl.BlockSpec((bm, h), lambda i: (i, 0)),
      ],
      out_shape=[
          jax.ShapeDtypeStruct((n, h), jnp.float32),
          jax.ShapeDtypeStruct((npad, h), jnp.float32),
      ],
  )(degp, xw0)


def _h1_heads(s1p, xw0, dinv, wc, npad, bm):
  n, h = xw0.shape

  def body(s_ref, x_ref, dv_ref, w_ref, c_ref, g_ref):
    dinv = dv_ref[...]
    agg = dinv * (s_ref[0] + s_ref[1]) + dinv * dinv * x_ref[...]
    h1 = jnp.maximum(agg, 0.0)
    c = jnp.dot(h1, w_ref[...], preferred_element_type=jnp.float32)
    c_ref[...] = c
    g_ref[...] = dinv * c

  return pl.pallas_call(
      body,
      grid=(n // bm,),
      in_specs=[
          pl.BlockSpec((2, bm, h), lambda i: (0, i, 0)),
          pl.BlockSpec((bm, h), lambda i: (i, 0)),
          pl.BlockSpec((bm, h), lambda i: (i, 0)),
          pl.BlockSpec((h, h), lambda i: (0, 0)),
      ],
      out_specs=[
          pl.BlockSpec((bm, h), lambda i: (i, 0)),
          pl.BlockSpec((bm, h), lambda i: (i, 0)),
      ],
      out_shape=[
          jax.ShapeDtypeStruct((n, h), jnp.float32),
          jax.ShapeDtypeStruct((npad, h), jnp.float32),
      ],
  )(s1p, xw0, dinv, wc)


def _reparam(s2p, c, dinv, eps, bm):
  n, h = c.shape
  h2 = h // 2

  def body(s_ref, c_ref, dv_ref, e_ref, z_ref):
    dinv = dv_ref[...]
    zc = dinv * (s_ref[0] + s_ref[1]) + dinv * dinv * c_ref[...]
    zm = zc[:, :h2]
    zl = zc[:, h2:]
    z_ref[...] = zm + e_ref[...] * jnp.exp(zl)

  return pl.pallas_call(
      body,
      grid=(n // bm,),
      in_specs=[
          pl.BlockSpec((2, bm, h), lambda i: (0, i, 0)),
          pl.BlockSpec((bm, h), lambda i: (i, 0)),
          pl.BlockSpec((bm, h), lambda i: (i, 0)),
          pl.BlockSpec((bm, h2), lambda i: (i, 0)),
      ],
      out_specs=pl.BlockSpec((bm, h2), lambda i: (i, 0)),
      out_shape=jax.ShapeDtypeStruct((n, h2), jnp.float32),
  )(s2p, c, dinv, eps)


def _decoder(z, zt, bm):
  n, k = z.shape

  def body(a_ref, b_ref, o_ref):
    m = jnp.dot(a_ref[...], b_ref[...], preferred_element_type=jnp.float32)
    for r in range(bm):
      o_ref[pl.ds(r * n, n)] = m[r, :]

  return pl.pallas_call(
      body,
      grid=(pl.cdiv(n, bm),),
      in_specs=[
          pl.BlockSpec((bm, k), lambda i: (i, 0)),
          pl.BlockSpec((k, n), lambda i: (0, 0)),
      ],
      out_specs=pl.BlockSpec((bm * n,), lambda i: (i,)),
      out_shape=jax.ShapeDtypeStruct((n * n,), jnp.float32),
  )(z, zt)


def kernel(x, edge_index, W0, W1, W2, eps):
  n = x.shape[0]
  e = edge_index.shape[1]

  npad = ((n + NS * 8 - 1) // (NS * 8)) * (NS * 8)
  unit = 4 * 4 * CH
  pair_units = ((e + NS * unit - 1) // (NS * unit) + 1) // 2 * 2
  u0 = max(2, min(pair_units - 2, round(pair_units / 5)))
  epw0, epw1 = u0 * unit, (pair_units - u0) * unit
  epad = NS * (epw0 + epw1)
  epw = epad // NW

  pad = epad - e
  src = jnp.concatenate([edge_index[0], jnp.full((pad,), n, jnp.int32)])
  dst = jnp.concatenate([edge_index[1], jnp.full((pad,), n, jnp.int32)])
  src = src.reshape(-1, CH)
  dst = dst.reshape(-1, CH)

  ones_blk = jnp.ones((CH, 16), jnp.float32)
  zrow16 = jnp.zeros((npad // NS, 16), jnp.float32)
  zrow32 = jnp.zeros((npad // NS, 32), jnp.float32)

  deg_k = _make_deg_kernel(npad, epw)
  agg_k = _make_agg_kernel(npad, epw0, epw1, 32, kc=4, nbuf=4)

  degp = deg_k(dst, ones_blk, zrow16).reshape(NC, npad, 16)

  xw0 = _matmul_xw0(x, W0, bm=1000)
  dinv, g0p = _prep_g0(degp, xw0, npad, bm=1000)

  s1p = agg_k(src, dst, g0p, zrow32).reshape(NC, npad, 32)

  wc = jnp.concatenate([W1, W2], axis=1)
  c, g1p = _h1_heads(s1p, xw0, dinv, wc, npad, bm=1000)

  s2p = agg_k(src, dst, g1p, zrow32).reshape(NC, npad, 32)

  z = _reparam(s2p, c, dinv, eps, bm=1000)

  return _decoder(z, z.T, bm=64)


# speedup vs baseline: 1.3281x; 1.3281x over previous
import functools

import jax
import jax.numpy as jnp
from jax import lax
from jax.experimental import pallas as pl
from jax.experimental.pallas import tpu as pltpu
from jax.experimental.pallas import tpu_sc as plsc

NC = 2
NS = 16
NW = NC * NS
CH = 128
K = 8


def _sc_mesh():
  return plsc.VectorSubcoreMesh(core_axis_name="c", subcore_axis_name="s")


def _make_deg_kernel(npad, epw):
  rpt = npad // NS
  w = 16
  nslabs = epw // (K * CH)
  assert nslabs % 2 == 0 and nslabs >= 2

  @functools.partial(
      pl.kernel,
      out_type=jax.ShapeDtypeStruct((NC * npad, w), jnp.float32),
      mesh=_sc_mesh(),
      scratch_types=[
          pltpu.VMEM((K, CH), jnp.int32),
          pltpu.VMEM((K, CH), jnp.int32),
          pltpu.VMEM((CH, w), jnp.float32),
          pltpu.VMEM_SHARED((npad, w), jnp.float32),
          pltpu.SemaphoreType.DMA,
          pltpu.SemaphoreType.DMA,
      ],
      compiler_params=pltpu.CompilerParams(use_tc_tiling_on_sc=False),
  )
  def deg_kernel(dst2_hbm, ones_hbm, zrow_hbm, out_hbm,
                 dst_s0, dst_s1, ones_v, acc, ssem0, ssem1):
    cid = lax.axis_index("c")
    sid = lax.axis_index("s")
    wid = sid * NC + cid
    r0 = wid * (epw // CH)

    pltpu.sync_copy(zrow_hbm, acc.at[pl.ds(sid * rpt, rpt)])
    pltpu.sync_copy(ones_hbm, ones_v)
    plsc.subcore_barrier()

    def load(buf, s):
      pltpu.sync_copy(dst2_hbm.at[pl.ds(r0 + s * K, K)], buf)

    def fire(buf, sem):
      for k in range(K):
        pltpu.async_copy(ones_v, acc.at[buf.at[k]], sem, add=True)

    def drain(sem):
      for _ in range(K):
        pltpu.make_async_copy(ones_hbm, ones_v, sem).wait()

    load(dst_s0, 0)

    def body(j, carry):
      s = 2 * j
      fire(dst_s0, ssem0)

      @pl.when(j > 0)
      def _():
        drain(ssem1)

      load(dst_s1, s + 1)
      fire(dst_s1, ssem1)
      drain(ssem0)

      @pl.when(j + 1 < nslabs // 2)
      def _():
        load(dst_s0, s + 2)

      return carry

    lax.fori_loop(0, nslabs // 2, body, 0)
    drain(ssem1)
    plsc.subcore_barrier()
    pltpu.sync_copy(acc.at[pl.ds(sid * rpt, rpt)],
                    out_hbm.at[pl.ds(cid * npad + sid * rpt, rpt)])

  return deg_kernel


def _make_agg_kernel(npad, epw0, epw1, w, kc, nbuf):
  rpt = npad // NS
  ns0 = epw0 // (kc * CH)
  ns1 = epw1 // (kc * CH)
  for nslabs in (ns0, ns1):
    assert nslabs % nbuf == 0 and nslabs >= 2 * nbuf

  @functools.partial(
      pl.kernel,
      out_type=jax.ShapeDtypeStruct((NC * npad, w), jnp.float32),
      mesh=_sc_mesh(),
      scratch_types=(
          [pltpu.VMEM((kc, CH), jnp.int32) for _ in range(nbuf)] +
          [pltpu.VMEM((kc, CH), jnp.int32) for _ in range(nbuf)] +
          [pltpu.VMEM((kc * CH, w), jnp.float32) for _ in range(nbuf)] +
          [pltpu.VMEM_SHARED((npad, w), jnp.float32)] +
          [pltpu.VMEM_SHARED((npad, w), jnp.float32)] +
          [pltpu.SemaphoreType.DMA for _ in range(2 * nbuf)]
      ),
      compiler_params=pltpu.CompilerParams(use_tc_tiling_on_sc=False),
  )
  def agg_kernel(src2_hbm, dst2_hbm, g_hbm, zrow_hbm, out_hbm, *bufs):
    src_s = bufs[0:nbuf]
    dst_s = bufs[nbuf:2 * nbuf]
    rows = bufs[2 * nbuf:3 * nbuf]
    acc = bufs[3 * nbuf]
    gbuf = bufs[3 * nbuf + 1]
    gsem = bufs[3 * nbuf + 2:3 * nbuf + 2 + nbuf]
    ssem = bufs[3 * nbuf + 2 + nbuf:3 * nbuf + 2 + 2 * nbuf]

    cid = lax.axis_index("c")
    sid = lax.axis_index("s")
    cw = jnp.where(cid == 0, epw0 // CH, epw1 // CH)
    nslabs = jnp.where(cid == 0, ns0, ns1)
    r0 = cid * (NS * (epw0 // CH)) + sid * cw

    pltpu.sync_copy(zrow_hbm, acc.at[pl.ds(sid * rpt, rpt)])
    pltpu.sync_copy(g_hbm.at[pl.ds(sid * rpt, rpt)],
                    gbuf.at[pl.ds(sid * rpt, rpt)])
    plsc.subcore_barrier()

    def fire_gathers(b, s):
      pltpu.sync_copy(src2_hbm.at[pl.ds(r0 + s * kc, kc)], src_s[b])
      pltpu.sync_copy(dst2_hbm.at[pl.ds(r0 + s * kc, kc)], dst_s[b])
      for k in range(kc):
        pltpu.async_copy(gbuf.at[src_s[b].at[k]],
                         rows[b].at[pl.ds(k * CH, CH)], gsem[b])

    def drain_gathers(b):
      pltpu.make_async_copy(g_hbm.at[pl.ds(0, kc * CH)], rows[b],
                            gsem[b]).wait()

    def fire_scatters(b):
      for k in range(kc):
        pltpu.async_copy(rows[b].at[pl.ds(k * CH, CH)],
                         acc.at[dst_s[b].at[k]], ssem[b], add=True)

    def drain_scatters(b):
      pltpu.make_async_copy(g_hbm.at[pl.ds(0, kc * CH)], rows[b],
                            ssem[b]).wait()

    for b in range(nbuf - 1):
      fire_gathers(b, b)

    def body(j, carry):
      for r in range(nbuf):
        s = nbuf * j + r
        b2 = (r + nbuf - 1) % nbuf

        @pl.when(s + nbuf - 1 < nslabs)
        def _():
          if r == 0:
            @pl.when(j > 0)
            def _():
              drain_scatters(b2)
          else:
            drain_scatters(b2)
          fire_gathers(b2, s + nbuf - 1)

        drain_gathers(r)
        fire_scatters(r)
      return carry

    lax.fori_loop(0, nslabs // nbuf, body, 0)
    for b in range(nbuf):
      drain_scatters(b)
    plsc.subcore_barrier()
    pltpu.sync_copy(acc.at[pl.ds(sid * rpt, rpt)],
                    out_hbm.at[pl.ds(cid * npad + sid * rpt, rpt)])

  return agg_kernel


def _matmul_xw0(x, w0, bm):
  n, d = x.shape
  h = w0.shape[1]

  def body(x_ref, w_ref, o_ref):
    o_ref[...] = jnp.dot(x_ref[...], w_ref[...],
                         preferred_element_type=jnp.float32)

  return pl.pallas_call(
      body,
      grid=(n // bm,),
      in_specs=[
          pl.BlockSpec((bm, d), lambda i: (i, 0)),
          pl.BlockSpec((d, h), lambda i: (0, 0)),
      ],
      out_specs=pl.BlockSpec((bm, h), lambda i: (i, 0)),
      out_shape=jax.ShapeDtypeStruct((n, h), jnp.float32),
  )(x, w0)


def _prep_g0(degp, xw0, npad, bm):
  n, h = xw0.shape

  def body(d_ref, x_ref, dinv_ref, g_ref):
    deg = d_ref[0, :, 0:1] + d_ref[1, :, 0:1] + 1.0
    dinv = lax.rsqrt(deg)
    dinv_b = jnp.broadcast_to(dinv, (bm, h))
    dinv_ref[...] = dinv_b
    g_ref[...] = dinv_b * x_ref[...]

  return pl.pallas_call(
      body,
      grid=(n // bm,),
      in_specs=[
          pl.BlockSpec((2, bm, 16), lambda i: (0, i, 0)),
          pl.BlockSpec((bm, h), lambda i: (i, 0)),
      ],
      out_specs=[
          pl.BlockSpec((bm, h), lambda i: (i, 0)),
          pl.BlockSpec((bm, h), lambda i: (i, 0)),
      ],
      out_shape=[
          jax.ShapeDtypeStruct((n, h), jnp.float32),
          jax.ShapeDtypeStruct((npad, h), jnp.float32),
      ],
  )(degp, xw0)


def _h1_heads(s1p, xw0, dinv, wc, npad, bm):
  n, h = xw0.shape

  def body(s_ref, x_ref, dv_ref, w_ref, c_ref, g_ref):
    dinv = dv_ref[...]
    agg = dinv * (s_ref[0] + s_ref[1]) + dinv * dinv * x_ref[...]
    h1 = jnp.maximum(agg, 0.0)
    c = jnp.dot(h1, w_ref[...], preferred_element_type=jnp.float32)
    c_ref[...] = c
    g_ref[...] = dinv * c

  return pl.pallas_call(
      body,
      grid=(n // bm,),
      in_specs=[
          pl.BlockSpec((2, bm, h), lambda i: (0, i, 0)),
          pl.BlockSpec((bm, h), lambda i: (i, 0)),
          pl.BlockSpec((bm, h), lambda i: (i, 0)),
          pl.BlockSpec((h, h), lambda i: (0, 0)),
      ],
      out_specs=[
          pl.BlockSpec((bm, h), lambda i: (i, 0)),
          pl.BlockSpec((bm, h), lambda i: (i, 0)),
      ],
      out_shape=[
          jax.ShapeDtypeStruct((n, h), jnp.float32),
          jax.ShapeDtypeStruct((npad, h), jnp.float32),
      ],
  )(s1p, xw0, dinv, wc)


def _reparam(s2p, c, dinv, eps, bm):
  n, h = c.shape
  h2 = h // 2

  def body(s_ref, c_ref, dv_ref, e_ref, z_ref):
    dinv = dv_ref[...]
    zc = dinv * (s_ref[0] + s_ref[1]) + dinv * dinv * c_ref[...]
    zm = zc[:, :h2]
    zl = zc[:, h2:]
    z_ref[...] = zm + e_ref[...] * jnp.exp(zl)

  return pl.pallas_call(
      body,
      grid=(n // bm,),
      in_specs=[
          pl.BlockSpec((2, bm, h), lambda i: (0, i, 0)),
          pl.BlockSpec((bm, h), lambda i: (i, 0)),
          pl.BlockSpec((bm, h), lambda i: (i, 0)),
          pl.BlockSpec((bm, h2), lambda i: (i, 0)),
      ],
      out_specs=pl.BlockSpec((bm, h2), lambda i: (i, 0)),
      out_shape=jax.ShapeDtypeStruct((n, h2), jnp.float32),
  )(s2p, c, dinv, eps)


def _decoder(z, zt, bm):
  n, k = z.shape

  def body(a_ref, b_ref, o_ref):
    m = jnp.dot(a_ref[...], b_ref[...], preferred_element_type=jnp.float32)
    for r in range(bm):
      o_ref[pl.ds(r * n, n)] = m[r, :]

  return pl.pallas_call(
      body,
      grid=(pl.cdiv(n, bm),),
      in_specs=[
          pl.BlockSpec((bm, k), lambda i: (i, 0)),
          pl.BlockSpec((k, n), lambda i: (0, 0)),
      ],
      out_specs=pl.BlockSpec((bm * n,), lambda i: (i,)),
      out_shape=jax.ShapeDtypeStruct((n * n,), jnp.float32),
  )(z, zt)


def kernel(x, edge_index, W0, W1, W2, eps):
  n = x.shape[0]
  e = edge_index.shape[1]

  npad = ((n + NS * 8 - 1) // (NS * 8)) * (NS * 8)
  unit = 4 * 4 * CH
  pair_units = ((e + NS * unit - 1) // (NS * unit) + 1) // 2 * 2
  u0 = pair_units // 2
  epw0, epw1 = u0 * unit, (pair_units - u0) * unit
  epad = NS * (epw0 + epw1)
  epw = epad // NW

  pad = epad - e
  src = jnp.concatenate([edge_index[0], jnp.full((pad,), n, jnp.int32)])
  dst = jnp.concatenate([edge_index[1], jnp.full((pad,), n, jnp.int32)])
  src = src.reshape(-1, CH)
  dst = dst.reshape(-1, CH)

  ones_blk = jnp.ones((CH, 16), jnp.float32)
  zrow16 = jnp.zeros((npad // NS, 16), jnp.float32)
  zrow32 = jnp.zeros((npad // NS, 32), jnp.float32)

  deg_k = _make_deg_kernel(npad, epw)
  agg_k = _make_agg_kernel(npad, epw0, epw1, 32, kc=4, nbuf=4)

  degp = deg_k(dst, ones_blk, zrow16).reshape(NC, npad, 16)

  xw0 = _matmul_xw0(x, W0, bm=1000)
  dinv, g0p = _prep_g0(degp, xw0, npad, bm=1000)

  s1p = agg_k(src, dst, g0p, zrow32).reshape(NC, npad, 32)

  wc = jnp.concatenate([W1, W2], axis=1)
  c, g1p = _h1_heads(s1p, xw0, dinv, wc, npad, bm=1000)

  s2p = agg_k(src, dst, g1p, zrow32).reshape(NC, npad, 32)

  z = _reparam(s2p, c, dinv, eps, bm=1000)

  return _decoder(z, z.T, bm=64)


# speedup vs baseline: 1.4334x; 1.0793x over previous
import functools

import jax
import jax.numpy as jnp
from jax import lax
from jax.experimental import pallas as pl
from jax.experimental.pallas import tpu as pltpu
from jax.experimental.pallas import tpu_sc as plsc

NC = 2
NS = 16
NW = NC * NS
CH = 128
K = 8


def _sc_mesh():
  return plsc.VectorSubcoreMesh(core_axis_name="c", subcore_axis_name="s")


def _make_deg_kernel(npad, epw):
  rpt = npad // NS
  w = 16
  nslabs = epw // (K * CH)
  assert nslabs % 2 == 0 and nslabs >= 2

  @functools.partial(
      pl.kernel,
      out_type=jax.ShapeDtypeStruct((NC, npad, w), jnp.float32),
      mesh=_sc_mesh(),
      scratch_types=[
          pltpu.VMEM((K, CH), jnp.int32),
          pltpu.VMEM((K, CH), jnp.int32),
          pltpu.VMEM((CH, w), jnp.float32),
          pltpu.VMEM_SHARED((npad, w), jnp.float32),
          pltpu.SemaphoreType.DMA,
          pltpu.SemaphoreType.DMA,
      ],
      compiler_params=pltpu.CompilerParams(use_tc_tiling_on_sc=False),
  )
  def deg_kernel(dst2_hbm, ones_hbm, zrow_hbm, out_hbm,
                 dst_s0, dst_s1, ones_v, acc, ssem0, ssem1):
    cid = lax.axis_index("c")
    sid = lax.axis_index("s")
    wid = sid * NC + cid
    r0 = wid * (epw // CH)

    pltpu.sync_copy(zrow_hbm, acc.at[pl.ds(sid * rpt, rpt)])
    pltpu.sync_copy(ones_hbm, ones_v)
    plsc.subcore_barrier()

    def load(buf, s):
      pltpu.sync_copy(dst2_hbm.at[pl.ds(r0 + s * K, K)], buf)

    def fire(buf, sem):
      for k in range(K):
        pltpu.async_copy(ones_v, acc.at[buf.at[k]], sem, add=True)

    def drain(sem):
      for _ in range(K):
        pltpu.make_async_copy(ones_hbm, ones_v, sem).wait()

    load(dst_s0, 0)

    def body(j, carry):
      s = 2 * j
      fire(dst_s0, ssem0)

      @pl.when(j > 0)
      def _():
        drain(ssem1)

      load(dst_s1, s + 1)
      fire(dst_s1, ssem1)
      drain(ssem0)

      @pl.when(j + 1 < nslabs // 2)
      def _():
        load(dst_s0, s + 2)

      return carry

    lax.fori_loop(0, nslabs // 2, body, 0)
    drain(ssem1)
    plsc.subcore_barrier()
    pltpu.sync_copy(acc.at[pl.ds(sid * rpt, rpt)],
                    out_hbm.at[cid, pl.ds(sid * rpt, rpt)])

  return deg_kernel


def _make_agg_kernel(npad, epw0, epw1, w, kc, nbuf):
  rpt = npad // NS
  ns0 = epw0 // (kc * CH)
  ns1 = epw1 // (kc * CH)
  for nslabs in (ns0, ns1):
    assert nslabs % nbuf == 0 and nslabs >= 2 * nbuf

  @functools.partial(
      pl.kernel,
      out_type=jax.ShapeDtypeStruct((NC, npad, w), jnp.float32),
      mesh=_sc_mesh(),
      scratch_types=(
          [pltpu.VMEM((kc, CH), jnp.int32) for _ in range(nbuf)] +
          [pltpu.VMEM((kc, CH), jnp.int32) for _ in range(nbuf)] +
          [pltpu.VMEM((kc * CH, w), jnp.float32) for _ in range(nbuf)] +
          [pltpu.VMEM_SHARED((npad, w), jnp.float32)] +
          [pltpu.VMEM_SHARED((npad, w), jnp.float32)] +
          [pltpu.SemaphoreType.DMA for _ in range(2 * nbuf)]
      ),
      compiler_params=pltpu.CompilerParams(use_tc_tiling_on_sc=False),
  )
  def agg_kernel(src2_hbm, dst2_hbm, g_hbm, zrow_hbm, out_hbm, *bufs):
    src_s = bufs[0:nbuf]
    dst_s = bufs[nbuf:2 * nbuf]
    rows = bufs[2 * nbuf:3 * nbuf]
    acc = bufs[3 * nbuf]
    gbuf = bufs[3 * nbuf + 1]
    gsem = bufs[3 * nbuf + 2:3 * nbuf + 2 + nbuf]
    ssem = bufs[3 * nbuf + 2 + nbuf:3 * nbuf + 2 + 2 * nbuf]

    cid = lax.axis_index("c")
    sid = lax.axis_index("s")
    cw = jnp.where(cid == 0, epw0 // CH, epw1 // CH)
    nslabs = jnp.where(cid == 0, ns0, ns1)
    r0 = cid * (NS * (epw0 // CH)) + sid * cw

    pltpu.sync_copy(zrow_hbm, acc.at[pl.ds(sid * rpt, rpt)])
    pltpu.sync_copy(g_hbm.at[pl.ds(sid * rpt, rpt)],
                    gbuf.at[pl.ds(sid * rpt, rpt)])
    plsc.subcore_barrier()

    def fire_gathers(b, s):
      pltpu.sync_copy(src2_hbm.at[pl.ds(r0 + s * kc, kc)], src_s[b])
      pltpu.sync_copy(dst2_hbm.at[pl.ds(r0 + s * kc, kc)], dst_s[b])
      for k in range(kc):
        pltpu.async_copy(gbuf.at[src_s[b].at[k]],
                         rows[b].at[pl.ds(k * CH, CH)], gsem[b])

    def drain_gathers(b):
      pltpu.make_async_copy(g_hbm.at[pl.ds(0, kc * CH)], rows[b],
                            gsem[b]).wait()

    def fire_scatters(b):
      for k in range(kc):
        pltpu.async_copy(rows[b].at[pl.ds(k * CH, CH)],
                         acc.at[dst_s[b].at[k]], ssem[b], add=True)

    def drain_scatters(b):
      pltpu.make_async_copy(g_hbm.at[pl.ds(0, kc * CH)], rows[b],
                            ssem[b]).wait()

    for b in range(nbuf - 1):
      fire_gathers(b, b)

    def body(j, carry):
      for r in range(nbuf):
        s = nbuf * j + r
        b2 = (r + nbuf - 1) % nbuf

        @pl.when(s + nbuf - 1 < nslabs)
        def _():
          if r == 0:
            @pl.when(j > 0)
            def _():
              drain_scatters(b2)
          else:
            drain_scatters(b2)
          fire_gathers(b2, s + nbuf - 1)

        drain_gathers(r)
        fire_scatters(r)
      return carry

    lax.fori_loop(0, nslabs // nbuf, body, 0)
    for b in range(nbuf):
      drain_scatters(b)
    plsc.subcore_barrier()
    pltpu.sync_copy(acc.at[pl.ds(sid * rpt, rpt)],
                    out_hbm.at[cid, pl.ds(sid * rpt, rpt)])

  return agg_kernel


def _matmul_xw0(x, w0, bm):
  n, d = x.shape
  h = w0.shape[1]

  def body(x_ref, w_ref, o_ref):
    o_ref[...] = jnp.dot(x_ref[...], w_ref[...],
                         preferred_element_type=jnp.float32)

  return pl.pallas_call(
      body,
      grid=(n // bm,),
      in_specs=[
          pl.BlockSpec((bm, d), lambda i: (i, 0)),
          pl.BlockSpec((d, h), lambda i: (0, 0)),
      ],
      out_specs=pl.BlockSpec((bm, h), lambda i: (i, 0)),
      out_shape=jax.ShapeDtypeStruct((n, h), jnp.float32),
  )(x, w0)


def _prep_g0(degp, xw0, npad, bm):
  n, h = xw0.shape

  def body(d_ref, x_ref, dinv_ref, g_ref):
    deg = d_ref[0, :, 0:1] + d_ref[1, :, 0:1] + 1.0
    dinv = lax.rsqrt(deg)
    dinv_b = jnp.broadcast_to(dinv, (bm, h))
    dinv_ref[...] = dinv_b
    g_ref[...] = dinv_b * x_ref[...]

  return pl.pallas_call(
      body,
      grid=(n // bm,),
      in_specs=[
          pl.BlockSpec((2, bm, 16), lambda i: (0, i, 0)),
          pl.BlockSpec((bm, h), lambda i: (i, 0)),
      ],
      out_specs=[
          pl.BlockSpec((bm, h), lambda i: (i, 0)),
          pl.BlockSpec((bm, h), lambda i: (i, 0)),
      ],
      out_shape=[
          jax.ShapeDtypeStruct((n, h), jnp.float32),
          jax.ShapeDtypeStruct((npad, h), jnp.float32),
      ],
  )(degp, xw0)


def _h1_heads(s1p, xw0, dinv, w1, w2, npad, bm):
  n, h = xw0.shape

  def body(s_ref, x_ref, dv_ref, w1_ref, w2_ref, c_ref, g_ref):
    dinv = dv_ref[...]
    agg = dinv * (s_ref[0] + s_ref[1]) + dinv * dinv * x_ref[...]
    h1 = jnp.maximum(agg, 0.0)
    w = jnp.concatenate([w1_ref[...], w2_ref[...]], axis=1)
    c = jnp.dot(h1, w, preferred_element_type=jnp.float32)
    c_ref[...] = c
    g_ref[...] = dinv * c

  h2 = w1.shape[1]
  return pl.pallas_call(
      body,
      grid=(n // bm,),
      in_specs=[
          pl.BlockSpec((2, bm, h), lambda i: (0, i, 0)),
          pl.BlockSpec((bm, h), lambda i: (i, 0)),
          pl.BlockSpec((bm, h), lambda i: (i, 0)),
          pl.BlockSpec((h, h2), lambda i: (0, 0)),
          pl.BlockSpec((h, h2), lambda i: (0, 0)),
      ],
      out_specs=[
          pl.BlockSpec((bm, h), lambda i: (i, 0)),
          pl.BlockSpec((bm, h), lambda i: (i, 0)),
      ],
      out_shape=[
          jax.ShapeDtypeStruct((n, h), jnp.float32),
          jax.ShapeDtypeStruct((npad, h), jnp.float32),
      ],
  )(s1p, xw0, dinv, w1, w2)


def _reparam(s2p, c, dinv, eps, bm):
  n, h = c.shape
  h2 = h // 2

  def body(s_ref, c_ref, dv_ref, e_ref, z_ref):
    dinv = dv_ref[...]
    zc = dinv * (s_ref[0] + s_ref[1]) + dinv * dinv * c_ref[...]
    zm = zc[:, :h2]
    zl = zc[:, h2:]
    z_ref[...] = zm + e_ref[...] * jnp.exp(zl)

  return pl.pallas_call(
      body,
      grid=(n // bm,),
      in_specs=[
          pl.BlockSpec((2, bm, h), lambda i: (0, i, 0)),
          pl.BlockSpec((bm, h), lambda i: (i, 0)),
          pl.BlockSpec((bm, h), lambda i: (i, 0)),
          pl.BlockSpec((bm, h2), lambda i: (i, 0)),
      ],
      out_specs=pl.BlockSpec((bm, h2), lambda i: (i, 0)),
      out_shape=jax.ShapeDtypeStruct((n, h2), jnp.float32),
  )(s2p, c, dinv, eps)


def _decoder(z, zt, bm):
  n, k = z.shape

  def body(a_ref, b_ref, o_ref):
    m = jnp.dot(a_ref[...], b_ref[...], preferred_element_type=jnp.float32)
    for r in range(bm):
      o_ref[pl.ds(r * n, n)] = m[r, :]

  return pl.pallas_call(
      body,
      grid=(pl.cdiv(n, bm),),
      in_specs=[
          pl.BlockSpec((bm, k), lambda i: (i, 0)),
          pl.BlockSpec((k, n), lambda i: (0, 0)),
      ],
      out_specs=pl.BlockSpec((bm * n,), lambda i: (i,)),
      out_shape=jax.ShapeDtypeStruct((n * n,), jnp.float32),
  )(z, zt)


def kernel(x, edge_index, W0, W1, W2, eps):
  n = x.shape[0]
  e = edge_index.shape[1]

  npad = ((n + NS * 8 - 1) // (NS * 8)) * (NS * 8)
  unit = 4 * 4 * CH
  pair_units = ((e + NS * unit - 1) // (NS * unit) + 1) // 2 * 2
  u0 = pair_units // 2
  epw0, epw1 = u0 * unit, (pair_units - u0) * unit
  epad = NS * (epw0 + epw1)
  epw = epad // NW

  pad = epad - e
  src = jnp.concatenate([edge_index[0], jnp.full((pad,), n, jnp.int32)])
  dst = jnp.concatenate([edge_index[1], jnp.full((pad,), n, jnp.int32)])
  src = src.reshape(-1, CH)
  dst = dst.reshape(-1, CH)

  ones_blk = jnp.ones((CH, 16), jnp.float32)
  zrow16 = jnp.zeros((npad // NS, 16), jnp.float32)
  zrow32 = jnp.zeros((npad // NS, 32), jnp.float32)

  deg_k = _make_deg_kernel(npad, epw)
  agg_k = _make_agg_kernel(npad, epw0, epw1, 32, kc=4, nbuf=4)

  degp = deg_k(dst, ones_blk, zrow16)

  xw0 = _matmul_xw0(x, W0, bm=1000)
  dinv, g0p = _prep_g0(degp, xw0, npad, bm=2000)

  s1p = agg_k(src, dst, g0p, zrow32)

  c, g1p = _h1_heads(s1p, xw0, dinv, W1, W2, npad, bm=2000)

  s2p = agg_k(src, dst, g1p, zrow32)

  z = _reparam(s2p, c, dinv, eps, bm=2000)

  return _decoder(z, z.T, bm=128)
